# baseline (speedup 1.0000x reference)
import functools

import jax
import jax.numpy as jnp
from jax import lax
from jax.experimental import pallas as pl
from jax.experimental.pallas import tpu as pltpu
from jax.experimental.pallas import tpu_sc as plsc


def _lookup_body(cam_hbm, idd_hbm, dt_hbm,
                 wcam1_hbm, bcam1_hbm, wident1_hbm, bident1_hbm,
                 wcam2_hbm, bcam2_hbm, wident2_hbm, bident2_hbm,
                 w_out_hbm, b_out_hbm,
                 cam_v, idd_v, dt_v, g_v, w_v, b_v, sem):
    wid = lax.axis_index("s") * 2 + lax.axis_index("c")

    @pl.when(wid == 0)
    def _():
        pltpu.sync_copy(cam_hbm, cam_v)
        pltpu.sync_copy(idd_hbm, idd_v)
        pltpu.sync_copy(dt_hbm, dt_v)
        cam = cam_v[...]
        idd = idd_v[...]
        tables = (wcam1_hbm, bcam1_hbm, wident1_hbm, bident1_hbm,
                  wcam2_hbm, bcam2_hbm, wident2_hbm, bident2_hbm)
        copies = []
        for t, tab in enumerate(tables):
            idx = (cam if t in (0, 1, 4, 5) else idd) * 3
            for c in range(3):
                copies.append(
                    pltpu.async_copy(tab.at[idx + c], g_v.at[t * 3 + c], sem))
        for cp in copies:
            cp.wait()

        use1 = dt_v[...] == 0
        for c in range(3):
            w1 = g_v[0 + c, :] + g_v[6 + c, :]
            b1 = g_v[3 + c, :] + g_v[9 + c, :]
            w2 = g_v[12 + c, :] + g_v[18 + c, :]
            b2 = g_v[15 + c, :] + g_v[21 + c, :]
            w_v[c, :] = jnp.where(use1, w1, w2)
            b_v[c, :] = jnp.where(use1, b1, b2)
        pltpu.sync_copy(w_v, w_out_hbm)
        pltpu.sync_copy(b_v, b_out_hbm)


def _sc_lookup(camindex, idindex, dataset_type,
               wcam1, bcam1, wident1, bident1,
               wcam2, bcam2, wident2, bident2):
    mesh = plsc.VectorSubcoreMesh(core_axis_name="c", subcore_axis_name="s")
    run = pl.kernel(
        _lookup_body,
        out_type=(jax.ShapeDtypeStruct((3, 16), jnp.float32),
                  jax.ShapeDtypeStruct((3, 16), jnp.float32)),
        mesh=mesh,
        scratch_types=[
            pltpu.VMEM((16,), jnp.int32),
            pltpu.VMEM((16,), jnp.int32),
            pltpu.VMEM((16,), jnp.int32),
            pltpu.VMEM((24, 16), jnp.float32),
            pltpu.VMEM((3, 16), jnp.float32),
            pltpu.VMEM((3, 16), jnp.float32),
            pltpu.SemaphoreType.DMA,
        ],
    )
    return run(camindex, idindex, dataset_type,
               wcam1.reshape(-1), bcam1.reshape(-1),
               wident1.reshape(-1), bident1.reshape(-1),
               wcam2.reshape(-1), bcam2.reshape(-1),
               wident2.reshape(-1), bident2.reshape(-1))


def _fused_body(cam_s, idd_s, dt_s, img_ref,
                wc1, bc1, wi1, bi1, wc2, bc2, wi2, bi2, o_ref):
    n = pl.program_id(0)
    cam = cam_s[n]
    idd = idd_s[n]
    use1 = dt_s[n] == 0
    w1 = wc1[pl.ds(cam, 1), :] + wi1[pl.ds(idd, 1), :]
    b1 = bc1[pl.ds(cam, 1), :] + bi1[pl.ds(idd, 1), :]
    w2 = wc2[pl.ds(cam, 1), :] + wi2[pl.ds(idd, 1), :]
    b2 = bc2[pl.ds(cam, 1), :] + bi2[pl.ds(idd, 1), :]
    w = jnp.where(use1, w1, w2)
    b = jnp.where(use1, b1, b2)
    for c in range(3):
        wv = lax.slice(w, (0, c), (1, c + 1))
        bv = lax.slice(b, (0, c), (1, c + 1))
        o_ref[0, c] = img_ref[0, c] * wv + bv


def _tc_fused(image, camindex, idindex, dataset_type, tables):
    n, ch, h, wd = image.shape
    smem = pl.BlockSpec(memory_space=pltpu.SMEM)
    full = pl.BlockSpec(memory_space=pltpu.VMEM)
    return pl.pallas_call(
        _fused_body,
        grid=(n,),
        in_specs=[smem, smem, smem,
                  pl.BlockSpec((1, ch, h, wd), lambda i: (i, 0, 0, 0)),
                  full, full, full, full, full, full, full, full],
        out_specs=pl.BlockSpec((1, ch, h, wd), lambda i: (i, 0, 0, 0)),
        out_shape=jax.ShapeDtypeStruct(image.shape, image.dtype),
        compiler_params=pltpu.CompilerParams(
            dimension_semantics=("parallel",)),
    )(camindex, idindex, dataset_type, image, *tables)


@jax.jit
def kernel(image, camindex, idindex, dataset_type,
           wcam1, bcam1, wident1, bident1,
           wcam2, bcam2, wident2, bident2):
    return _tc_fused(image, camindex, idindex, dataset_type,
                     (wcam1, bcam1, wident1, bident1,
                      wcam2, bcam2, wident2, bident2))


# speedup vs baseline: 1.1826x; 1.1826x over previous
import functools

import jax
import jax.numpy as jnp
from jax import lax
from jax.experimental import pallas as pl
from jax.experimental.pallas import tpu as pltpu
from jax.experimental.pallas import tpu_sc as plsc


def _lookup_body(cam_hbm, idd_hbm, dt_hbm,
                 wcam1_hbm, bcam1_hbm, wident1_hbm, bident1_hbm,
                 wcam2_hbm, bcam2_hbm, wident2_hbm, bident2_hbm,
                 w_out_hbm, b_out_hbm,
                 cam_v, idd_v, dt_v, g_v, w_v, b_v, sem):
    wid = lax.axis_index("s") * 2 + lax.axis_index("c")

    @pl.when(wid == 0)
    def _():
        pltpu.sync_copy(cam_hbm, cam_v)
        pltpu.sync_copy(idd_hbm, idd_v)
        pltpu.sync_copy(dt_hbm, dt_v)
        cam = cam_v[...]
        idd = idd_v[...]
        tables = (wcam1_hbm, bcam1_hbm, wident1_hbm, bident1_hbm,
                  wcam2_hbm, bcam2_hbm, wident2_hbm, bident2_hbm)
        copies = []
        for t, tab in enumerate(tables):
            idx = (cam if t in (0, 1, 4, 5) else idd) * 3
            for c in range(3):
                copies.append(
                    pltpu.async_copy(tab.at[idx + c], g_v.at[t * 3 + c], sem))
        for cp in copies:
            cp.wait()

        use1 = dt_v[...] == 0
        for c in range(3):
            w1 = g_v[0 + c, :] + g_v[6 + c, :]
            b1 = g_v[3 + c, :] + g_v[9 + c, :]
            w2 = g_v[12 + c, :] + g_v[18 + c, :]
            b2 = g_v[15 + c, :] + g_v[21 + c, :]
            w_v[c, :] = jnp.where(use1, w1, w2)
            b_v[c, :] = jnp.where(use1, b1, b2)
        pltpu.sync_copy(w_v, w_out_hbm)
        pltpu.sync_copy(b_v, b_out_hbm)


def _sc_lookup(camindex, idindex, dataset_type,
               wcam1, bcam1, wident1, bident1,
               wcam2, bcam2, wident2, bident2):
    mesh = plsc.VectorSubcoreMesh(core_axis_name="c", subcore_axis_name="s")
    run = pl.kernel(
        _lookup_body,
        out_type=(jax.ShapeDtypeStruct((3, 16), jnp.float32),
                  jax.ShapeDtypeStruct((3, 16), jnp.float32)),
        mesh=mesh,
        scratch_types=[
            pltpu.VMEM((16,), jnp.int32),
            pltpu.VMEM((16,), jnp.int32),
            pltpu.VMEM((16,), jnp.int32),
            pltpu.VMEM((24, 16), jnp.float32),
            pltpu.VMEM((3, 16), jnp.float32),
            pltpu.VMEM((3, 16), jnp.float32),
            pltpu.SemaphoreType.DMA,
        ],
    )
    return run(camindex, idindex, dataset_type,
               wcam1.reshape(-1), bcam1.reshape(-1),
               wident1.reshape(-1), bident1.reshape(-1),
               wcam2.reshape(-1), bcam2.reshape(-1),
               wident2.reshape(-1), bident2.reshape(-1))


_NB = 2


def _affine_body(img_ref, w_ref, b_ref, o_ref):
    n0 = pl.program_id(0) * _NB
    for k in range(_NB):
        for c in range(3):
            o_ref[k, c] = img_ref[k, c] * w_ref[c, n0 + k] + b_ref[c, n0 + k]


def _tc_affine(image, w2d, b2d):
    n, ch, h, wd = image.shape
    return pl.pallas_call(
        _affine_body,
        grid=(n // _NB,),
        in_specs=[
            pl.BlockSpec((_NB, ch, h, wd), lambda i: (i, 0, 0, 0)),
            pl.BlockSpec(memory_space=pltpu.SMEM),
            pl.BlockSpec(memory_space=pltpu.SMEM),
        ],
        out_specs=pl.BlockSpec((_NB, ch, h, wd), lambda i: (i, 0, 0, 0)),
        out_shape=jax.ShapeDtypeStruct(image.shape, image.dtype),
        compiler_params=pltpu.CompilerParams(
            dimension_semantics=("parallel",)),
    )(image, w2d, b2d)


@jax.jit
def kernel(image, camindex, idindex, dataset_type,
           wcam1, bcam1, wident1, bident1,
           wcam2, bcam2, wident2, bident2):
    w1 = jnp.take(wcam1, camindex, axis=0) + jnp.take(wident1, idindex, axis=0)
    b1 = jnp.take(bcam1, camindex, axis=0) + jnp.take(bident1, idindex, axis=0)
    w2 = jnp.take(wcam2, camindex, axis=0) + jnp.take(wident2, idindex, axis=0)
    b2 = jnp.take(bcam2, camindex, axis=0) + jnp.take(bident2, idindex, axis=0)
    mask = (dataset_type == 0)[:, None]
    w2d = jnp.where(mask, w1, w2).T
    b2d = jnp.where(mask, b1, b2).T
    return _tc_affine(image, w2d, b2d)


# speedup vs baseline: 1.2127x; 1.0254x over previous
import functools

import jax
import jax.numpy as jnp
from jax import lax
from jax.experimental import pallas as pl
from jax.experimental.pallas import tpu as pltpu
from jax.experimental.pallas import tpu_sc as plsc


def _lookup_body(cam_hbm, idd_hbm, dt_hbm,
                 wcam1_hbm, bcam1_hbm, wident1_hbm, bident1_hbm,
                 wcam2_hbm, bcam2_hbm, wident2_hbm, bident2_hbm,
                 w_out_hbm, b_out_hbm,
                 cam_v, idd_v, dt_v, g_v, w_v, b_v, sem):
    wid = lax.axis_index("s") * 2 + lax.axis_index("c")

    @pl.when(wid == 0)
    def _():
        pltpu.sync_copy(cam_hbm, cam_v)
        pltpu.sync_copy(idd_hbm, idd_v)
        pltpu.sync_copy(dt_hbm, dt_v)
        cam = cam_v[...]
        idd = idd_v[...]
        tables = (wcam1_hbm, bcam1_hbm, wident1_hbm, bident1_hbm,
                  wcam2_hbm, bcam2_hbm, wident2_hbm, bident2_hbm)
        copies = []
        for t, tab in enumerate(tables):
            idx = (cam if t in (0, 1, 4, 5) else idd) * 3
            for c in range(3):
                copies.append(
                    pltpu.async_copy(tab.at[idx + c], g_v.at[t * 3 + c], sem))
        for cp in copies:
            cp.wait()

        use1 = dt_v[...] == 0
        for c in range(3):
            w1 = g_v[0 + c, :] + g_v[6 + c, :]
            b1 = g_v[3 + c, :] + g_v[9 + c, :]
            w2 = g_v[12 + c, :] + g_v[18 + c, :]
            b2 = g_v[15 + c, :] + g_v[21 + c, :]
            w_v[c, :] = jnp.where(use1, w1, w2)
            b_v[c, :] = jnp.where(use1, b1, b2)
        pltpu.sync_copy(w_v, w_out_hbm)
        pltpu.sync_copy(b_v, b_out_hbm)


def _sc_lookup(camindex, idindex, dataset_type,
               wcam1, bcam1, wident1, bident1,
               wcam2, bcam2, wident2, bident2):
    mesh = plsc.VectorSubcoreMesh(core_axis_name="c", subcore_axis_name="s")
    run = pl.kernel(
        _lookup_body,
        out_type=(jax.ShapeDtypeStruct((3, 16), jnp.float32),
                  jax.ShapeDtypeStruct((3, 16), jnp.float32)),
        mesh=mesh,
        scratch_types=[
            pltpu.VMEM((16,), jnp.int32),
            pltpu.VMEM((16,), jnp.int32),
            pltpu.VMEM((16,), jnp.int32),
            pltpu.VMEM((24, 16), jnp.float32),
            pltpu.VMEM((3, 16), jnp.float32),
            pltpu.VMEM((3, 16), jnp.float32),
            pltpu.SemaphoreType.DMA,
        ],
    )
    return run(camindex, idindex, dataset_type,
               wcam1.reshape(-1), bcam1.reshape(-1),
               wident1.reshape(-1), bident1.reshape(-1),
               wcam2.reshape(-1), bcam2.reshape(-1),
               wident2.reshape(-1), bident2.reshape(-1))


_NB = 4


def _affine_body(img_ref, w_ref, b_ref, o_ref):
    n0 = pl.program_id(0) * _NB
    for k in range(_NB):
        for c in range(3):
            o_ref[k, c] = img_ref[k, c] * w_ref[c, n0 + k] + b_ref[c, n0 + k]


def _tc_affine(image, w2d, b2d):
    n, ch, h, wd = image.shape
    return pl.pallas_call(
        _affine_body,
        grid=(n // _NB,),
        in_specs=[
            pl.BlockSpec((_NB, ch, h, wd), lambda i: (i, 0, 0, 0)),
            pl.BlockSpec(memory_space=pltpu.SMEM),
            pl.BlockSpec(memory_space=pltpu.SMEM),
        ],
        out_specs=pl.BlockSpec((_NB, ch, h, wd), lambda i: (i, 0, 0, 0)),
        out_shape=jax.ShapeDtypeStruct(image.shape, image.dtype),
        compiler_params=pltpu.CompilerParams(
            dimension_semantics=("parallel",)),
    )(image, w2d, b2d)


@jax.jit
def kernel(image, camindex, idindex, dataset_type,
           wcam1, bcam1, wident1, bident1,
           wcam2, bcam2, wident2, bident2):
    w1 = jnp.take(wcam1, camindex, axis=0) + jnp.take(wident1, idindex, axis=0)
    b1 = jnp.take(bcam1, camindex, axis=0) + jnp.take(bident1, idindex, axis=0)
    w2 = jnp.take(wcam2, camindex, axis=0) + jnp.take(wident2, idindex, axis=0)
    b2 = jnp.take(bcam2, camindex, axis=0) + jnp.take(bident2, idindex, axis=0)
    mask = (dataset_type == 0)[:, None]
    w2d = jnp.where(mask, w1, w2).T
    b2d = jnp.where(mask, b1, b2).T
    return _tc_affine(image, w2d, b2d)
